# baseline (speedup 1.0000x reference)
import jax
import jax.numpy as jnp
from jax import lax
from jax.experimental import pallas as pl

_N = 10000
_E = 320000
_D = 128
_DE = 16
_NCONV = 3
_H = 128
_G = 100

_NPAD = 10240
_NBLK = _NPAD // 128
_EBLK = _E // 128


def _embed_body(nf_ref, emb_ref, x_ref):
    idx = nf_ref[0, 0:1, :]
    iota = lax.broadcasted_iota(jnp.int32, (128, 128), 1)
    oh = (idx.T == iota).astype(jnp.float32)
    x_ref[...] = jnp.dot(oh, emb_ref[...],
                         preferred_element_type=jnp.float32)


def _tc_embed(nf3d, emb_pad):
    return pl.pallas_call(
        _embed_body,
        grid=(_NBLK,),
        in_specs=[
            pl.BlockSpec((1, 8, 128), lambda i: (i, 0, 0)),
            pl.BlockSpec((128, _D), lambda i: (0, 0)),
        ],
        out_specs=pl.BlockSpec((128, _D), lambda i: (i, 0)),
        out_shape=jax.ShapeDtypeStruct((_NPAD, _D), jnp.float32),
    )(nf3d, emb_pad)


def _edge_affine_body(i1_ref, i2_ref, ef_ref, x_ref, w1t_ref, w2t_ref,
                      w3t_ref, b_ref, z_ref, st_ref):
    i = pl.program_id(0)

    @pl.when(i == 0)
    def _init():
        st_ref[...] = jnp.zeros_like(st_ref)

    iota = lax.broadcasted_iota(jnp.int32, (128, _NPAD), 1)
    oh1 = (i1_ref[0, 0:1, :].T == iota).astype(jnp.float32)
    oh2 = (i2_ref[0, 0:1, :].T == iota).astype(jnp.float32)
    n1 = jnp.dot(oh1, x_ref[...], preferred_element_type=jnp.float32)
    n2 = jnp.dot(oh2, x_ref[...], preferred_element_type=jnp.float32)
    blk = (jnp.dot(n1, w1t_ref[...], preferred_element_type=jnp.float32)
           + jnp.dot(n2, w2t_ref[...], preferred_element_type=jnp.float32)
           + jnp.dot(ef_ref[...], w3t_ref[...],
                     preferred_element_type=jnp.float32)
           + b_ref[0:1, :])
    z_ref[...] = blk
    st_ref[0:1, :] += jnp.sum(blk, axis=0, keepdims=True)
    st_ref[1:2, :] += jnp.sum(blk * blk, axis=0, keepdims=True)


def _tc_edge_affine(i1b, i2b, ef, x, w1t, w2t, w3t, b_blk):
    return pl.pallas_call(
        _edge_affine_body,
        grid=(_EBLK,),
        in_specs=[
            pl.BlockSpec((1, 8, 128), lambda i: (i, 0, 0)),
            pl.BlockSpec((1, 8, 128), lambda i: (i, 0, 0)),
            pl.BlockSpec((128, _DE), lambda i: (i, 0)),
            pl.BlockSpec((_NPAD, _D), lambda i: (0, 0)),
            pl.BlockSpec((_D, 2 * _D), lambda i: (0, 0)),
            pl.BlockSpec((_D, 2 * _D), lambda i: (0, 0)),
            pl.BlockSpec((_DE, 2 * _D), lambda i: (0, 0)),
            pl.BlockSpec((8, 2 * _D), lambda i: (0, 0)),
        ],
        out_specs=[
            pl.BlockSpec((128, 2 * _D), lambda i: (i, 0)),
            pl.BlockSpec((8, 2 * _D), lambda i: (0, 0)),
        ],
        out_shape=[
            jax.ShapeDtypeStruct((_E, 2 * _D), jnp.float32),
            jax.ShapeDtypeStruct((8, 2 * _D), jnp.float32),
        ],
    )(i1b, i2b, ef, x, w1t, w2t, w3t, b_blk)


def _edge_msg_body(z_ref, st_ref, g_ref, b_ref, msg_ref):
    m = st_ref[0:1, :] / _E
    v = st_ref[1:2, :] / _E - m * m
    scale = g_ref[0:1, :] / jnp.sqrt(v + 1e-5)
    shift = b_ref[0:1, :] - scale * m
    zb = z_ref[...] * scale + shift
    gate = jax.nn.sigmoid(zb[:, :_D])
    conv = jax.nn.softplus(zb[:, _D:])
    msg_ref[...] = gate * conv


def _tc_edge_msg(z, stats, g_blk, b_blk):
    return pl.pallas_call(
        _edge_msg_body,
        grid=(_EBLK,),
        in_specs=[
            pl.BlockSpec((128, 2 * _D), lambda i: (i, 0)),
            pl.BlockSpec((8, 2 * _D), lambda i: (0, 0)),
            pl.BlockSpec((8, 2 * _D), lambda i: (0, 0)),
            pl.BlockSpec((8, 2 * _D), lambda i: (0, 0)),
        ],
        out_specs=pl.BlockSpec((128, _D), lambda i: (i, 0)),
        out_shape=jax.ShapeDtypeStruct((_E, _D), jnp.float32),
    )(z, stats, g_blk, b_blk)


def _agg_body(idx_ref, msg_ref, acc_ref, cnt_ref):
    i = pl.program_id(0)

    @pl.when(i == 0)
    def _init():
        acc_ref[...] = jnp.zeros_like(acc_ref)
        cnt_ref[...] = jnp.zeros_like(cnt_ref)

    idx = idx_ref[0, 0:1, :]
    iota = lax.broadcasted_iota(jnp.int32, (128, _NPAD), 1)
    oh = (idx.T == iota).astype(jnp.float32)
    acc_ref[...] += lax.dot_general(
        oh, msg_ref[...], (((0,), (0,)), ((), ())),
        preferred_element_type=jnp.float32)
    cnt_ref[0:1, :] += jnp.sum(oh, axis=0, keepdims=True)


def _tc_agg(i1b, msg):
    return pl.pallas_call(
        _agg_body,
        grid=(_EBLK,),
        in_specs=[
            pl.BlockSpec((1, 8, 128), lambda i: (i, 0, 0)),
            pl.BlockSpec((128, _D), lambda i: (i, 0)),
        ],
        out_specs=[
            pl.BlockSpec((_NPAD, _D), lambda i: (0, 0)),
            pl.BlockSpec((8, _NPAD), lambda i: (0, 0)),
        ],
        out_shape=[
            jax.ShapeDtypeStruct((_NPAD, _D), jnp.float32),
            jax.ShapeDtypeStruct((8, _NPAD), jnp.float32),
        ],
    )(i1b, msg)


def _node_body(x_ref, p0_ref, cb_ref, g_ref, b_ref, out_ref):
    agg = p0_ref[...] / cb_ref[...]
    m = jnp.sum(agg, axis=0, keepdims=True) / _N
    v = jnp.sum((agg - m) ** 2, axis=0, keepdims=True) / _N
    aggn = g_ref[0:1, :] * (agg - m) / jnp.sqrt(v + 1e-5) + b_ref[0:1, :]
    out_ref[...] = jax.nn.softplus(x_ref[...] + aggn)


def _tc_node(x, p0, cb, g_blk, b_blk):
    return pl.pallas_call(
        _node_body,
        out_shape=jax.ShapeDtypeStruct((_N, _D), jnp.float32),
    )(x, p0, cb, g_blk, b_blk)


def _readout_body(x_ref, i3_ref, w1t_ref, b1_ref, wot_ref, bo_ref, out_ref):
    iota = lax.broadcasted_iota(jnp.int32, (_N, _D), 1)
    oh = (i3_ref[...] == iota).astype(jnp.float32)
    csum = lax.dot_general(oh, x_ref[...], (((0,), (0,)), ((), ())),
                           preferred_element_type=jnp.float32)
    cnt = jnp.maximum(jnp.sum(oh, axis=0, keepdims=True), 1.0)
    crys = csum / cnt.T
    h = jax.nn.softplus(
        jnp.dot(crys, w1t_ref[...], preferred_element_type=jnp.float32)
        + b1_ref[0:1, :])
    out_ref[...] = (jnp.dot(h, wot_ref[...], preferred_element_type=jnp.float32)
                    + bo_ref[0:1, :])


def _tc_readout(x, i3b, w1t, b1_blk, wot, bo_blk):
    return pl.pallas_call(
        _readout_body,
        out_shape=jax.ShapeDtypeStruct((_D, _D), jnp.float32),
    )(x, i3b, w1t, b1_blk, wot, bo_blk)


def kernel(node_fea, edge_fea, idx1, idx2, idx3, emb, fc_full_W, fc_full_b,
           bn1_g, bn1_b, bn2_g, bn2_b, fc1_W, fc1_b, out_W, out_b):
    f32 = jnp.float32
    nf3d = jnp.broadcast_to(
        jnp.pad(node_fea.astype(jnp.int32),
                (0, _NPAD - _N)).reshape(_NBLK, 1, 128),
        (_NBLK, 8, 128))
    i1b = jnp.broadcast_to(
        idx1.astype(jnp.int32).reshape(_EBLK, 1, 128), (_EBLK, 8, 128))
    i2b = jnp.broadcast_to(
        idx2.astype(jnp.int32).reshape(_EBLK, 1, 128), (_EBLK, 8, 128))
    ef = edge_fea.astype(f32)
    emb_pad = jnp.pad(emb.astype(f32), ((0, 128 - emb.shape[0]), (0, 0)))

    x = _tc_embed(nf3d, emb_pad)
    cb = None

    for i in range(_NCONV):
        W = fc_full_W[i]
        w1t = W[:, :_D].T
        w2t = W[:, _D:2 * _D].T
        w3t = W[:, 2 * _D:].T
        b_blk = jnp.broadcast_to(fc_full_b[i][None, :], (8, 2 * _D))
        g1 = jnp.broadcast_to(bn1_g[i][None, :], (8, 2 * _D))
        b1 = jnp.broadcast_to(bn1_b[i][None, :], (8, 2 * _D))
        g2 = jnp.broadcast_to(bn2_g[i][None, :], (8, _D))
        b2 = jnp.broadcast_to(bn2_b[i][None, :], (8, _D))

        z, stats = _tc_edge_affine(i1b, i2b, ef, x, w1t, w2t, w3t, b_blk)
        msg = _tc_edge_msg(z, stats, g1, b1)
        aggs, cnts = _tc_agg(i1b, msg)
        if cb is None:
            c = cnts[0, :_N]
            cb = jnp.broadcast_to(jnp.maximum(c, 1.0)[:, None], (_N, _D))
        xn = _tc_node(x[:_N], aggs[:_N], cb, g2, b2)
        x = jnp.pad(xn, ((0, _NPAD - _N), (0, 0)))

    i3b = jnp.broadcast_to(idx3.astype(jnp.int32)[:, None], (_N, _D))
    w1t_r = fc1_W.T
    b1_r = jnp.broadcast_to(fc1_b[None, :], (8, _H))
    wot = jnp.zeros((_H, _D), f32).at[:, :2].set(out_W.T)
    bo = jnp.zeros((8, _D), f32).at[:, :2].set(
        jnp.broadcast_to(out_b[None, :], (8, 2)))
    outm = _tc_readout(x[:_N], i3b, w1t_r, b1_r, wot, bo)
    return outm[:_G, :2]
